# baseline (speedup 1.0000x reference)
import jax
import jax.numpy as jnp
from jax import lax
from jax.experimental import pallas as pl
from jax.experimental.pallas import tpu as pltpu
from jax.experimental.pallas import tpu_sc as plsc

D_MODEL = 768
SEQ = 8192
BATCH = 4
NC, NS, L = 2, 16, 16
NW = NC * NS
ROWS = BATCH * SEQ
RPW = ROWS // NW
G = 64
NCHUNK = RPW // G


def _sc_body(mask_hbm, table_hbm, out_hbm, idx_v, buf_v, gsem, ssem):
    wid = lax.axis_index("s") * NC + lax.axis_index("c")
    base = wid * RPW
    jbase = base % SEQ

    for c in range(NCHUNK):
        pltpu.sync_copy(mask_hbm.at[pl.ds(base + c * G, G)], idx_v.at[c])

    iota = lax.iota(jnp.int32, L)
    for c in range(NCHUNK):
        for k in range(G // L):
            m = idx_v[c, pl.ds(k * L, L)]
            pos = (jbase + c * G + k * L + 1) + iota
            idx_v[c, pl.ds(k * L, L)] = pos * m

    def gather(c):
        return pltpu.make_async_copy(
            table_hbm.at[idx_v.at[c]], buf_v.at[c % 2], gsem.at[c % 2])

    def scatter(c):
        return pltpu.make_async_copy(
            buf_v.at[c % 2], out_hbm.at[pl.ds(base + c * G, G)], ssem.at[c % 2])

    for c in range(NCHUNK):
        if c >= 2:
            scatter(c - 2).wait()
        gather(c).start()
        gather(c).wait()
        scatter(c).start()
    scatter(NCHUNK - 2).wait()
    scatter(NCHUNK - 1).wait()


def kernel(input_ids, mask, table):
    del input_ids
    mask_flat = mask.reshape(ROWS).astype(jnp.int32)
    table = table.astype(jnp.float32)

    out = pl.kernel(
        _sc_body,
        out_type=jax.ShapeDtypeStruct((ROWS, D_MODEL), jnp.float32),
        mesh=plsc.VectorSubcoreMesh(core_axis_name="c", subcore_axis_name="s"),
        scratch_types=[
            pltpu.VMEM((NCHUNK, G), jnp.int32),
            pltpu.VMEM((2, G, D_MODEL), jnp.float32),
            pltpu.SemaphoreType.DMA((2,)),
            pltpu.SemaphoreType.DMA((2,)),
        ],
    )(mask_flat, table)
    return out.reshape(BATCH, SEQ, D_MODEL)


# speedup vs baseline: 1.9895x; 1.9895x over previous
import jax
import jax.numpy as jnp
from jax import lax
from jax.experimental import pallas as pl
from jax.experimental.pallas import tpu as pltpu
from jax.experimental.pallas import tpu_sc as plsc

D_MODEL = 768
SEQ = 8192
BATCH = 4
NC, NS, L = 2, 16, 16
NW = NC * NS
JC = SEQ // NW
S = 32
NSUB = JC // S
NV = D_MODEL // L
SD = S * D_MODEL

_GDN = lax.GatherDimensionNumbers(
    offset_dims=(), collapsed_slice_dims=(0,), start_index_map=(0,))


def _bcast_lane(vec, lane):
    idx = jnp.broadcast_to(lane, (L,)).astype(jnp.int32)[:, None]
    return lax.gather(vec, idx, _GDN, (1,),
                      mode=lax.GatherScatterMode.PROMISE_IN_BOUNDS)


def _sc_body(mask_hbm, table_hbm, out_hbm, mask_v, maskf_v, tbl_v, out_v, ssem):
    wid = lax.axis_index("s") * NC + lax.axis_index("c")
    jbase = wid * JC

    for b in range(BATCH):
        pltpu.sync_copy(mask_hbm.at[pl.ds(b * SEQ + jbase, JC)],
                        mask_v.at[pl.ds(b * JC, JC)])
    for k in range(BATCH * JC // L):
        maskf_v[pl.ds(k * L, L)] = mask_v[pl.ds(k * L, L)].astype(jnp.float32)

    def scat(b, sub):
        return pltpu.make_async_copy(
            out_v.at[b % 2],
            out_hbm.at[pl.ds((b * SEQ + jbase + sub * S) * D_MODEL, SD)],
            ssem.at[b % 2])

    def sub_body(sub, carry):
        pltpu.sync_copy(
            table_hbm.at[pl.ds((1 + jbase + sub * S) * D_MODEL, SD)], tbl_v)
        for b in range(BATCH):
            slot = b % 2

            def row_body(r, c):
                g = r & ~(L - 1)
                lane = r & (L - 1)
                mvec = maskf_v[pl.ds(b * JC + sub * S + g, L)]
                bm = _bcast_lane(mvec, lane)
                for v in range(NV):
                    out_v[slot, pl.ds(r * D_MODEL + v * L, L)] = (
                        tbl_v[pl.ds(r * D_MODEL + v * L, L)] * bm)
                return c

            lax.fori_loop(0, S, row_body, 0)
            scat(b, sub).start()
            if b == 0:
                @pl.when(sub > 0)
                def _():
                    scat(BATCH - 1, sub - 1).wait()
            else:
                scat(b - 1, sub).wait()
        return carry

    lax.fori_loop(0, NSUB, sub_body, 0)
    scat(BATCH - 1, NSUB - 1).wait()


def kernel(input_ids, mask, table):
    del input_ids
    mask_flat = mask.reshape(BATCH * SEQ).astype(jnp.int32)
    table_flat = table.astype(jnp.float32).reshape((SEQ + 1) * D_MODEL)

    out = pl.kernel(
        _sc_body,
        out_type=jax.ShapeDtypeStruct((BATCH * SEQ * D_MODEL,), jnp.float32),
        mesh=plsc.VectorSubcoreMesh(core_axis_name="c", subcore_axis_name="s"),
        scratch_types=[
            pltpu.VMEM((BATCH * JC,), jnp.int32),
            pltpu.VMEM((BATCH * JC,), jnp.float32),
            pltpu.VMEM((SD,), jnp.float32),
            pltpu.VMEM((2, SD), jnp.float32),
            pltpu.SemaphoreType.DMA((2,)),
        ],
    )(mask_flat, table_flat)
    return out.reshape(BATCH, SEQ, D_MODEL)


# speedup vs baseline: 1.9995x; 1.0050x over previous
import jax
import jax.numpy as jnp
from jax import lax
from jax.experimental import pallas as pl
from jax.experimental.pallas import tpu as pltpu
from jax.experimental.pallas import tpu_sc as plsc

D_MODEL = 768
SEQ = 8192
BATCH = 4
NC, NS, L = 2, 16, 16
NW = NC * NS
JC = SEQ // NW
S = 16
NSUB = JC // S
NV = D_MODEL // L
SD = S * D_MODEL

_GDN = lax.GatherDimensionNumbers(
    offset_dims=(), collapsed_slice_dims=(0,), start_index_map=(0,))


def _bcast_lane(vec, lane):
    idx = jnp.broadcast_to(lane, (L,)).astype(jnp.int32)[:, None]
    return lax.gather(vec, idx, _GDN, (1,),
                      mode=lax.GatherScatterMode.PROMISE_IN_BOUNDS)


def _sc_body(mask_hbm, table_hbm, out_hbm, mask_v, maskf_v, tbl_v, out_v,
             gsem, ssem):
    wid = lax.axis_index("s") * NC + lax.axis_index("c")
    jbase = wid * JC

    for b in range(BATCH):
        pltpu.sync_copy(mask_hbm.at[pl.ds(b * SEQ + jbase, JC)],
                        mask_v.at[pl.ds(b * JC, JC)])
    for k in range(BATCH * JC // L):
        maskf_v[pl.ds(k * L, L)] = mask_v[pl.ds(k * L, L)].astype(jnp.float32)

    def gath(sub, tslot):
        return pltpu.make_async_copy(
            table_hbm.at[pl.ds((1 + jbase + sub * S) * D_MODEL, SD)],
            tbl_v.at[tslot], gsem.at[tslot])

    def scat(b, sub):
        return pltpu.make_async_copy(
            out_v.at[b],
            out_hbm.at[pl.ds((b * SEQ + jbase + sub * S) * D_MODEL, SD)],
            ssem.at[b])

    def process(sub, tslot):
        for b in range(BATCH):
            @pl.when(sub > 0)
            def _():
                scat(b, sub - 1).wait()

            def row_body(r, c):
                mvec = maskf_v[pl.ds(b * JC + sub * S, L)]
                bm = _bcast_lane(mvec, r)
                for v in range(NV):
                    out_v[b, pl.ds(r * D_MODEL + v * L, L)] = (
                        tbl_v[tslot, pl.ds(r * D_MODEL + v * L, L)] * bm)
                return c

            lax.fori_loop(0, S, row_body, 0)
            scat(b, sub).start()

    gath(0, 0).start()

    def pair_body(it, carry):
        sub0 = 2 * it
        gath(sub0, 0).wait()
        gath(sub0 + 1, 1).start()
        process(sub0, 0)
        gath(sub0 + 1, 1).wait()

        @pl.when(it + 1 < NSUB // 2)
        def _():
            gath(sub0 + 2, 0).start()
        process(sub0 + 1, 1)
        return carry

    lax.fori_loop(0, NSUB // 2, pair_body, 0)
    for b in range(BATCH):
        scat(b, NSUB - 1).wait()


def kernel(input_ids, mask, table):
    del input_ids
    mask_flat = mask.reshape(BATCH * SEQ).astype(jnp.int32)
    table_flat = table.astype(jnp.float32).reshape((SEQ + 1) * D_MODEL)

    out = pl.kernel(
        _sc_body,
        out_type=jax.ShapeDtypeStruct((BATCH * SEQ * D_MODEL,), jnp.float32),
        mesh=plsc.VectorSubcoreMesh(core_axis_name="c", subcore_axis_name="s"),
        scratch_types=[
            pltpu.VMEM((BATCH * JC,), jnp.int32),
            pltpu.VMEM((BATCH * JC,), jnp.float32),
            pltpu.VMEM((2, SD), jnp.float32),
            pltpu.VMEM((BATCH, SD), jnp.float32),
            pltpu.SemaphoreType.DMA((2,)),
            pltpu.SemaphoreType.DMA((BATCH,)),
        ],
    )(mask_flat, table_flat)
    return out.reshape(BATCH, SEQ, D_MODEL)


# speedup vs baseline: 3.6760x; 1.8385x over previous
import jax
import jax.numpy as jnp
from jax import lax
from jax.experimental import pallas as pl
from jax.experimental.pallas import tpu as pltpu
from jax.experimental.pallas import tpu_sc as plsc

D_MODEL = 768
SEQ = 8192
BATCH = 4
NC, NS, L = 2, 16, 16
NW = NC * NS
JC = SEQ // NW
S = 16
NSUB = JC // S
NV = D_MODEL // L
SD = S * D_MODEL
HD = SD // 2

_GDN = lax.GatherDimensionNumbers(
    offset_dims=(), collapsed_slice_dims=(0,), start_index_map=(0,))


def _bcast_lane(vec, lane):
    idx = jnp.broadcast_to(lane, (L,)).astype(jnp.int32)[:, None]
    return lax.gather(vec, idx, _GDN, (1,),
                      mode=lax.GatherScatterMode.PROMISE_IN_BOUNDS)


def _sc_body(mask_hbm, table_hbm, out_hbm, mask_v, maskf_v, tbl_v, out_v,
             gsem, ssem):
    wid = lax.axis_index("s") * NC + lax.axis_index("c")
    jbase = wid * JC

    for b in range(BATCH):
        pltpu.sync_copy(mask_hbm.at[pl.ds(b * SEQ + jbase, JC)],
                        mask_v.at[pl.ds(b * JC, JC)])
    for k in range(BATCH * JC // L):
        maskf_v[pl.ds(k * L, L)] = mask_v[pl.ds(k * L, L)].astype(jnp.float32)

    def gath_parts(sub, tslot):
        off = (1 + jbase + sub * S) * D_MODEL
        return [
            pltpu.make_async_copy(table_hbm.at[pl.ds(off + h * HD, HD)],
                                  tbl_v.at[tslot, pl.ds(h * HD, HD)],
                                  gsem.at[2 * tslot + h])
            for h in range(2)
        ]

    def scat_parts(b, sub):
        off = (b * SEQ + jbase + sub * S) * D_MODEL
        return [
            pltpu.make_async_copy(out_v.at[b, pl.ds(h * HD, HD)],
                                  out_hbm.at[pl.ds(off + h * HD, HD)],
                                  ssem.at[2 * b + h])
            for h in range(2)
        ]

    def process(sub, tslot):
        for b in range(BATCH):
            @pl.when(sub > 0)
            def _():
                for p in scat_parts(b, sub - 1):
                    p.wait()

        mv = [maskf_v[pl.ds(b * JC + sub * S, L)] for b in range(BATCH)]

        @plsc.parallel_loop(0, S, 1, unroll=2)
        def _rows(r):
            bms = [_bcast_lane(mv[b], r) for b in range(BATCH)]
            for v in range(NV):
                t = tbl_v[tslot, pl.ds(r * D_MODEL + v * L, L)]
                for b in range(BATCH):
                    out_v[b, pl.ds(r * D_MODEL + v * L, L)] = t * bms[b]

        for b in range(BATCH):
            for p in scat_parts(b, sub):
                p.start()

    for p in gath_parts(0, 0):
        p.start()

    def pair_body(it, carry):
        sub0 = 2 * it
        for p in gath_parts(sub0, 0):
            p.wait()
        for p in gath_parts(sub0 + 1, 1):
            p.start()
        process(sub0, 0)
        for p in gath_parts(sub0 + 1, 1):
            p.wait()

        @pl.when(it + 1 < NSUB // 2)
        def _():
            for p in gath_parts(sub0 + 2, 0):
                p.start()
        process(sub0 + 1, 1)
        return carry

    lax.fori_loop(0, NSUB // 2, pair_body, 0)
    for b in range(BATCH):
        for p in scat_parts(b, NSUB - 1):
            p.wait()


def kernel(input_ids, mask, table):
    del input_ids
    mask_flat = mask.reshape(BATCH * SEQ).astype(jnp.int32)
    table_flat = table.astype(jnp.float32).reshape((SEQ + 1) * D_MODEL)

    out = pl.kernel(
        _sc_body,
        out_type=jax.ShapeDtypeStruct((BATCH * SEQ * D_MODEL,), jnp.float32),
        mesh=plsc.VectorSubcoreMesh(core_axis_name="c", subcore_axis_name="s"),
        scratch_types=[
            pltpu.VMEM((BATCH * JC,), jnp.int32),
            pltpu.VMEM((BATCH * JC,), jnp.float32),
            pltpu.VMEM((2, SD), jnp.float32),
            pltpu.VMEM((BATCH, SD), jnp.float32),
            pltpu.SemaphoreType.DMA((4,)),
            pltpu.SemaphoreType.DMA((2 * BATCH,)),
        ],
    )(mask_flat, table_flat)
    return out.reshape(BATCH, SEQ, D_MODEL)


# speedup vs baseline: 7.9694x; 2.1680x over previous
import jax
import jax.numpy as jnp
from jax import lax
from jax.experimental import pallas as pl
from jax.experimental.pallas import tpu as pltpu
from jax.experimental.pallas import tpu_sc as plsc

D_MODEL = 768
SEQ = 8192
BATCH = 4
NC, NS, L = 2, 16, 16
NW = NC * NS
JC = SEQ // NW
S = 16
NSUB = JC // S
NV = D_MODEL // L
W = S + 8
WMAX = SEQ + 1 - W

_GDN = lax.GatherDimensionNumbers(
    offset_dims=(), collapsed_slice_dims=(0,), start_index_map=(0,))


def _bcast_lane(vec, lane):
    idx = jnp.broadcast_to(lane, (L,)).astype(jnp.int32)[:, None]
    return lax.gather(vec, idx, _GDN, (1,),
                      mode=lax.GatherScatterMode.PROMISE_IN_BOUNDS)


def _sc_body(mask_hbm, table_hbm, out_hbm, mask_v, maskf_v, tbl_v, out_v,
             gsem, ssem):
    wid = lax.axis_index("s") * NC + lax.axis_index("c")
    jbase = wid * JC

    for b in range(BATCH):
        pltpu.sync_copy(mask_hbm.at[pl.ds(b * SEQ + jbase, JC)],
                        mask_v.at[pl.ds(b * JC, JC)])
    for k in range(BATCH * JC // L):
        maskf_v[pl.ds(k * L, L)] = mask_v[pl.ds(k * L, L)].astype(jnp.float32)

    def wstart(sub):
        j0 = jbase + sub * S
        return jnp.minimum(j0, (WMAX // 8) * 8)

    def gath(sub, tslot):
        return pltpu.make_async_copy(
            table_hbm.at[pl.ds(wstart(sub), W)], tbl_v.at[tslot],
            gsem.at[tslot])

    def scat_parts(b, sub):
        row0 = b * SEQ + jbase + sub * S
        return [
            pltpu.make_async_copy(out_v.at[b, pl.ds(h * (S // 2), S // 2)],
                                  out_hbm.at[pl.ds(row0 + h * (S // 2), S // 2)],
                                  ssem.at[2 * b + h])
            for h in range(2)
        ]

    def process(sub, tslot):
        for b in range(BATCH):
            @pl.when(sub > 0)
            def _():
                for p in scat_parts(b, sub - 1):
                    p.wait()

        mv = [maskf_v[pl.ds(b * JC + sub * S, L)] for b in range(BATCH)]
        roff = jbase + sub * S + 1 - wstart(sub)

        @plsc.parallel_loop(0, S, 1, unroll=2)
        def _rows(r):
            bms = [_bcast_lane(mv[b], r) for b in range(BATCH)]
            for v in range(NV):
                t = tbl_v[tslot, roff + r, pl.ds(v * L, L)]
                for b in range(BATCH):
                    out_v[b, r, pl.ds(v * L, L)] = t * bms[b]

        for b in range(BATCH):
            for p in scat_parts(b, sub):
                p.start()

    gath(0, 0).start()

    def pair_body(it, carry):
        sub0 = 2 * it
        gath(sub0, 0).wait()
        gath(sub0 + 1, 1).start()
        process(sub0, 0)
        gath(sub0 + 1, 1).wait()

        @pl.when(it + 1 < NSUB // 2)
        def _():
            gath(sub0 + 2, 0).start()
        process(sub0 + 1, 1)
        return carry

    lax.fori_loop(0, NSUB // 2, pair_body, 0)
    for b in range(BATCH):
        for p in scat_parts(b, NSUB - 1):
            p.wait()


def kernel(input_ids, mask, table):
    del input_ids
    mask_flat = mask.reshape(BATCH * SEQ).astype(jnp.int32)
    table = table.astype(jnp.float32)

    out = pl.kernel(
        _sc_body,
        out_type=jax.ShapeDtypeStruct((BATCH * SEQ, D_MODEL), jnp.float32),
        mesh=plsc.VectorSubcoreMesh(core_axis_name="c", subcore_axis_name="s"),
        scratch_types=[
            pltpu.VMEM((BATCH * JC,), jnp.int32),
            pltpu.VMEM((BATCH * JC,), jnp.float32),
            pltpu.VMEM((2, W, D_MODEL), jnp.float32),
            pltpu.VMEM((BATCH, S, D_MODEL), jnp.float32),
            pltpu.SemaphoreType.DMA((2,)),
            pltpu.SemaphoreType.DMA((2 * BATCH,)),
        ],
    )(mask_flat, table)
    return out.reshape(BATCH, SEQ, D_MODEL)
